# baseline (speedup 1.0000x reference)
import functools

import jax
import jax.numpy as jnp
from jax import lax
from jax.experimental import pallas as pl
from jax.experimental.pallas import tpu as pltpu
from jax.experimental.pallas import tpu_sc as plsc

D = 768
H = 12
DH = 64
E = 8
S = 2048
SBLK = 256
EPAD = 128
NEG = -1e30
QSCALE = 0.125 * 1.4426950408889634
NW = 32
CHUNK = S // NW


def _routing_body(lg_hbm, gates_hbm, lg_v, gt_v):
    wid = lax.axis_index("s") * 2 + lax.axis_index("c")
    base = wid * CHUNK
    for e in range(E):
        pltpu.sync_copy(lg_hbm.at[pl.ds(e * S + base, CHUNK)],
                        lg_v.at[pl.ds(e * CHUNK, CHUNK)])
    negv = jnp.full((16,), NEG, jnp.float32)
    zerov = jnp.full((16,), 0.0, jnp.float32)
    onev = jnp.full((16,), 1.0, jnp.float32)
    ev = [jnp.full((16,), e, jnp.int32) for e in range(E)]
    for c in range(CHUNK // 16):
        ls = [lg_v[pl.ds(e * CHUNK + c * 16, 16)] for e in range(E)]
        m1 = ls[0]
        for e in range(1, E):
            m1 = jnp.maximum(m1, ls[e])
        i1 = ev[0]
        for e in range(E - 1, -1, -1):
            i1 = jnp.where(ls[e] == m1, ev[e], i1)
        m2 = negv
        for e in range(E):
            m2 = jnp.maximum(m2, jnp.where(i1 == ev[e], negv, ls[e]))
        i2 = ev[0]
        for e in range(E - 1, -1, -1):
            le = jnp.where(i1 == ev[e], negv, ls[e])
            i2 = jnp.where(le == m2, ev[e], i2)
        eb = jnp.exp(m2 - m1)
        g1 = onev / (onev + eb)
        g2 = onev - g1
        for e in range(E):
            ge = (jnp.where(i1 == ev[e], g1, zerov)
                  + jnp.where(i2 == ev[e], g2, zerov))
            gt_v[pl.ds(e * CHUNK + c * 16, 16)] = ge
    for e in range(E):
        pltpu.sync_copy(gt_v.at[pl.ds(e * CHUNK, CHUNK)],
                        gates_hbm.at[pl.ds(e * S + base, CHUNK)])


def _sc_routing(logits_flat):
    mesh = plsc.VectorSubcoreMesh(core_axis_name="c", subcore_axis_name="s")
    fn = pl.kernel(
        _routing_body, mesh=mesh,
        out_type=jax.ShapeDtypeStruct((S * E,), jnp.float32),
        scratch_types=[pltpu.VMEM((CHUNK * E,), jnp.float32),
                       pltpu.VMEM((CHUNK * E,), jnp.float32)],
    )
    return fn(logits_flat)


def _ln_qkv_body(x_ref, wq_ref, bq_ref, wk_ref, bk_ref, wv_ref, bv_ref,
                 rw_ref, rb_ref, g_ref, b_ref,
                 q_ref, k_ref, v_ref, lg_ref):
    xv = x_ref[...]
    mu = jnp.mean(xv, axis=1, keepdims=True)
    xc = xv - mu
    var = jnp.mean(xc * xc, axis=1, keepdims=True)
    xn = xc * jax.lax.rsqrt(var + 1e-5) * g_ref[...] + b_ref[...]
    xb = xn.astype(jnp.bfloat16)
    q = jnp.dot(xb, wq_ref[...].astype(jnp.bfloat16),
                preferred_element_type=jnp.float32) + bq_ref[...]
    q_ref[...] = (q * QSCALE).astype(jnp.bfloat16)
    k_ref[...] = (jnp.dot(xb, wk_ref[...].astype(jnp.bfloat16),
                          preferred_element_type=jnp.float32)
                  + bk_ref[...]).astype(jnp.bfloat16)
    v_ref[...] = (jnp.dot(xb, wv_ref[...].astype(jnp.bfloat16),
                          preferred_element_type=jnp.float32)
                  + bv_ref[...]).astype(jnp.bfloat16)
    lg_ref[...] = jnp.dot(xn, rw_ref[...], preferred_element_type=jnp.float32) + rb_ref[...]


def _attn_body(q_ref, k_ref, v_ref, ctx_ref):
    qq = q_ref[...]
    kk = k_ref[...]
    vv = v_ref[...]
    ones = jnp.ones((S, DH), jnp.bfloat16)
    outs = []
    for h in range(2):
        q = qq[:, h * DH:(h + 1) * DH]
        k = kk[:, h * DH:(h + 1) * DH]
        v = vv[:, h * DH:(h + 1) * DH]
        s = jax.lax.dot_general(q, k, (((1,), (1,)), ((), ())),
                                preferred_element_type=jnp.float32)
        p = jnp.exp2(s.astype(jnp.bfloat16))
        c = jnp.dot(p, jnp.concatenate([v, ones], axis=1),
                    preferred_element_type=jnp.float32)
        outs.append((c[:, :DH] / c[:, DH:DH + 1]).astype(jnp.bfloat16))
    ctx_ref[...] = jnp.concatenate(outs, axis=1)


def _moe_body(ctx_ref, g_ref, x_ref, we_ref, be_ref, out_ref):
    gates = g_ref[...]
    ctx = ctx_ref[...]
    acc = x_ref[...]
    for e in range(E):
        ge = gates[:, e:e + 1]
        acc = acc + ge * (jnp.dot(ctx, we_ref[e].astype(jnp.bfloat16),
                                  preferred_element_type=jnp.float32)
                          + be_ref[e:e + 1, :])
    out_ref[...] = acc


def kernel(x, W_q, b_q, W_k, b_k, W_v, b_v, router_w, router_b,
           expert_w, expert_b, ln_gamma, ln_beta):
    xf = x.reshape(S, D)
    rw = jnp.pad(router_w, ((0, 0), (0, EPAD - E)))
    rb = jnp.pad(router_b, (0, EPAD - E)).reshape(1, EPAD)
    full = lambda *shape: pl.BlockSpec(shape, lambda i: (0,) * len(shape))
    row_blk = pl.BlockSpec((SBLK, D), lambda i: (i, 0))

    q, k, v, logits = pl.pallas_call(
        _ln_qkv_body,
        grid=(S // SBLK,),
        in_specs=[row_blk, full(D, D), full(1, D), full(D, D), full(1, D),
                  full(D, D), full(1, D), full(D, EPAD), full(1, EPAD),
                  full(1, D), full(1, D)],
        out_specs=[row_blk, row_blk, row_blk,
                   pl.BlockSpec((SBLK, EPAD), lambda i: (i, 0))],
        out_shape=[jax.ShapeDtypeStruct((S, D), jnp.bfloat16)] * 3
        + [jax.ShapeDtypeStruct((S, EPAD), jnp.float32)],
    )(xf, W_q, b_q.reshape(1, D), W_k, b_k.reshape(1, D),
      W_v, b_v.reshape(1, D), rw, rb,
      ln_gamma.reshape(1, D), ln_beta.reshape(1, D))

    head_blk = pl.BlockSpec((S, 2 * DH), lambda g: (0, g))
    ctx = pl.pallas_call(
        _attn_body,
        grid=(H // 2,),
        in_specs=[head_blk, head_blk, head_blk],
        out_specs=head_blk,
        out_shape=jax.ShapeDtypeStruct((S, D), jnp.bfloat16),
    )(q, k, v)

    gates = _sc_routing(logits[:, :E].T.reshape(E * S)).reshape(E, S).T

    out = pl.pallas_call(
        _moe_body,
        grid=(S // SBLK,),
        in_specs=[row_blk, pl.BlockSpec((SBLK, E), lambda i: (i, 0)),
                  row_blk, full(E, D, D), full(E, D)],
        out_specs=row_blk,
        out_shape=jax.ShapeDtypeStruct((S, D), jnp.float32),
    )(ctx, gates, xf, expert_w, expert_b)

    return out.reshape(1, S, D)


# speedup vs baseline: 1.0417x; 1.0417x over previous
import functools

import jax
import jax.numpy as jnp
from jax import lax
from jax.experimental import pallas as pl
from jax.experimental.pallas import tpu as pltpu
from jax.experimental.pallas import tpu_sc as plsc

D = 768
H = 12
DH = 64
E = 8
S = 2048
SBLK = 256
EPAD = 128
NEG = -1e30
QSCALE = 0.125 * 1.4426950408889634
NW = 32
CHUNK = S // NW


def _routing_body(lg_hbm, gates_hbm, lg_v, gt_v):
    wid = lax.axis_index("s") * 2 + lax.axis_index("c")
    base = wid * CHUNK
    for e in range(E):
        pltpu.sync_copy(lg_hbm.at[e, pl.ds(base, CHUNK)],
                        lg_v.at[pl.ds(e * CHUNK, CHUNK)])
    negv = jnp.full((16,), NEG, jnp.float32)
    zerov = jnp.full((16,), 0.0, jnp.float32)
    onev = jnp.full((16,), 1.0, jnp.float32)
    ev = [jnp.full((16,), e, jnp.int32) for e in range(E)]
    for c in range(CHUNK // 16):
        ls = [lg_v[pl.ds(e * CHUNK + c * 16, 16)] for e in range(E)]
        m1 = ls[0]
        for e in range(1, E):
            m1 = jnp.maximum(m1, ls[e])
        i1 = ev[0]
        for e in range(E - 1, -1, -1):
            i1 = jnp.where(ls[e] == m1, ev[e], i1)
        m2 = negv
        for e in range(E):
            m2 = jnp.maximum(m2, jnp.where(i1 == ev[e], negv, ls[e]))
        i2 = ev[0]
        for e in range(E - 1, -1, -1):
            le = jnp.where(i1 == ev[e], negv, ls[e])
            i2 = jnp.where(le == m2, ev[e], i2)
        eb = jnp.exp(m2 - m1)
        g1 = onev / (onev + eb)
        g2 = onev - g1
        for e in range(E):
            ge = (jnp.where(i1 == ev[e], g1, zerov)
                  + jnp.where(i2 == ev[e], g2, zerov))
            gt_v[pl.ds(e * CHUNK + c * 16, 16)] = ge
    for e in range(E):
        pltpu.sync_copy(gt_v.at[pl.ds(e * CHUNK, CHUNK)],
                        gates_hbm.at[e, pl.ds(base, CHUNK)])


def _sc_routing(logits_flat):
    mesh = plsc.VectorSubcoreMesh(core_axis_name="c", subcore_axis_name="s")
    fn = pl.kernel(
        _routing_body, mesh=mesh,
        out_type=jax.ShapeDtypeStruct((E, S), jnp.float32),
        scratch_types=[pltpu.VMEM((CHUNK * E,), jnp.float32),
                       pltpu.VMEM((CHUNK * E,), jnp.float32)],
    )
    return fn(logits_flat)


def _ln_qkv_body(x_ref, wq_ref, bq_ref, wk_ref, bk_ref, wv_ref, bv_ref,
                 rw_ref, rb_ref, g_ref, b_ref,
                 q_ref, k_ref, v_ref, lg_ref):
    xv = x_ref[...]
    mu = jnp.mean(xv, axis=1, keepdims=True)
    xc = xv - mu
    var = jnp.mean(xc * xc, axis=1, keepdims=True)
    xn = xc * jax.lax.rsqrt(var + 1e-5) * g_ref[...] + b_ref[...]
    xb = xn.astype(jnp.bfloat16)
    q = jnp.dot(xb, wq_ref[...].astype(jnp.bfloat16),
                preferred_element_type=jnp.float32) + bq_ref[...]
    q_ref[...] = (q * QSCALE).astype(jnp.bfloat16)
    k_ref[...] = (jnp.dot(xb, wk_ref[...].astype(jnp.bfloat16),
                          preferred_element_type=jnp.float32)
                  + bk_ref[...]).astype(jnp.bfloat16)
    v_ref[...] = (jnp.dot(xb, wv_ref[...].astype(jnp.bfloat16),
                          preferred_element_type=jnp.float32)
                  + bv_ref[...]).astype(jnp.bfloat16)
    lg_ref[...] = jax.lax.dot_general(
        rw_ref[...], xn, (((0,), (1,)), ((), ())),
        preferred_element_type=jnp.float32) + rb_ref[...]


def _attn_body(q_ref, k_ref, v_ref, ctx_ref):
    qq = q_ref[...]
    kk = k_ref[...]
    vv = v_ref[...]
    ones = jnp.ones((S, DH), jnp.bfloat16)
    outs = []
    for h in range(2):
        q = qq[:, h * DH:(h + 1) * DH]
        k = kk[:, h * DH:(h + 1) * DH]
        v = vv[:, h * DH:(h + 1) * DH]
        s = jax.lax.dot_general(q, k, (((1,), (1,)), ((), ())),
                                preferred_element_type=jnp.float32)
        p = jnp.exp2(s.astype(jnp.bfloat16))
        c = jnp.dot(p, jnp.concatenate([v, ones], axis=1),
                    preferred_element_type=jnp.float32)
        outs.append((c[:, :DH] / c[:, DH:DH + 1]).astype(jnp.bfloat16))
    ctx_ref[...] = jnp.concatenate(outs, axis=1)


def _moe_body(ctx_ref, g_ref, x_ref, we_ref, be_ref, out_ref):
    gates = jax.lax.dot_general(g_ref[...], jnp.eye(E, dtype=jnp.float32),
                                (((0,), (0,)), ((), ())),
                                preferred_element_type=jnp.float32)
    ctx = ctx_ref[...]
    acc = x_ref[...]
    for e in range(E):
        ge = gates[:, e:e + 1]
        acc = acc + ge * (jnp.dot(ctx, we_ref[e].astype(jnp.bfloat16),
                                  preferred_element_type=jnp.float32)
                          + be_ref[e:e + 1, :])
    out_ref[...] = acc


def kernel(x, W_q, b_q, W_k, b_k, W_v, b_v, router_w, router_b,
           expert_w, expert_b, ln_gamma, ln_beta):
    xf = x.reshape(S, D)
    full = lambda *shape: pl.BlockSpec(shape, lambda i: (0,) * len(shape))
    row_blk = pl.BlockSpec((SBLK, D), lambda i: (i, 0))

    q, k, v, logits = pl.pallas_call(
        _ln_qkv_body,
        grid=(S // SBLK,),
        in_specs=[row_blk, full(D, D), full(1, D), full(D, D), full(1, D),
                  full(D, D), full(1, D), full(D, E), full(E, 1),
                  full(1, D), full(1, D)],
        out_specs=[row_blk, row_blk, row_blk,
                   pl.BlockSpec((E, SBLK), lambda i: (0, i))],
        out_shape=[jax.ShapeDtypeStruct((S, D), jnp.bfloat16)] * 3
        + [jax.ShapeDtypeStruct((E, S), jnp.float32)],
    )(xf, W_q, b_q.reshape(1, D), W_k, b_k.reshape(1, D),
      W_v, b_v.reshape(1, D), router_w, router_b.reshape(E, 1),
      ln_gamma.reshape(1, D), ln_beta.reshape(1, D))

    head_blk = pl.BlockSpec((S, 2 * DH), lambda g: (0, g))
    ctx = pl.pallas_call(
        _attn_body,
        grid=(H // 2,),
        in_specs=[head_blk, head_blk, head_blk],
        out_specs=head_blk,
        out_shape=jax.ShapeDtypeStruct((S, D), jnp.bfloat16),
    )(q, k, v)

    gates = _sc_routing(logits)

    out = pl.pallas_call(
        _moe_body,
        grid=(S // SBLK,),
        in_specs=[row_blk, pl.BlockSpec((E, SBLK), lambda i: (0, i)),
                  row_blk, full(E, D, D), full(E, D)],
        out_specs=row_blk,
        out_shape=jax.ShapeDtypeStruct((S, D), jnp.float32),
    )(ctx, gates, xf, expert_w, expert_b)

    return out.reshape(1, S, D)
